# Initial kernel scaffold; baseline (speedup 1.0000x reference)
import jax
import jax.numpy as jnp
from jax.experimental import pallas as pl


def kernel(boxes, scores):
    raise NotImplementedError("write your pallas kernel here")


# speedup vs baseline: 35.2812x; 35.2812x over previous
import jax
import jax.numpy as jnp
from jax.experimental import pallas as pl
from jax.experimental.pallas import tpu as pltpu

_SCORE_THRESH = 0.05
_NMS_THRESH = 0.6
_PRE = 1000
_POST = 256
_CLAMP = 1e-4
_N = 1024
_BLK = 256


def _nms_body(vt_ref, v8_ref, out_ref, cu_ref):
    x1i = vt_ref[0:1, :]
    y1i = vt_ref[1:2, :]
    x2i = vt_ref[2:3, :]
    y2i = vt_ref[3:4, :]
    vali = vt_ref[4:5, :]
    area_i = (x2i - x1i) * (y2i - y1i)
    iidx = jax.lax.broadcasted_iota(jnp.int32, (_BLK, _N), 1)

    def fill_block(jb, carry):
        j0 = jb * _BLK
        bj = v8_ref[pl.ds(j0, _BLK), :]
        x1j = bj[:, 0:1]
        y1j = bj[:, 1:2]
        x2j = bj[:, 2:3]
        y2j = bj[:, 3:4]
        area_j = (x2j - x1j) * (y2j - y1j)
        xx1 = jnp.maximum(x1j, x1i)
        yy1 = jnp.maximum(y1j, y1i)
        xx2 = jnp.minimum(x2j, x2i)
        yy2 = jnp.minimum(y2j, y2i)
        w = jnp.maximum(xx2 - xx1, 0.0)
        h = jnp.maximum(yy2 - yy1, 0.0)
        inter = w * h
        union = area_j + area_i - inter
        iou = inter / jnp.maximum(union, 1e-6)
        jidx = j0 + jax.lax.broadcasted_iota(jnp.int32, (_BLK, _N), 0)
        cu = jnp.where((iou > _NMS_THRESH) & (jidx < iidx), 1.0, 0.0)
        cu_ref[pl.ds(j0, _BLK), :] = cu
        return carry

    jax.lax.fori_loop(0, _N // _BLK, fill_block, 0)

    cu = cu_ref[:]
    valid = jnp.where(vali > _SCORE_THRESH, 1.0, 0.0)

    def cond(c):
        return jnp.logical_not(c[1])

    def body(c):
        k, _ = c
        sup = jnp.dot(k, cu, preferred_element_type=jnp.float32)
        kn = jnp.where(sup > 0.0, 0.0, valid)
        return kn, jnp.all(kn == k)

    k, _ = jax.lax.while_loop(cond, body, (valid, jnp.array(False)))

    r = jax.lax.broadcasted_iota(jnp.int32, (_N, _N), 0)
    c = jax.lax.broadcasted_iota(jnp.int32, (_N, _N), 1)
    tri = jnp.where(r <= c, 1.0, 0.0)
    cumk = jnp.dot(k, tri, preferred_element_type=jnp.float32)
    pos = cumk - 1.0
    jout = jax.lax.broadcasted_iota(jnp.int32, (_POST, _N), 0).astype(jnp.float32)
    onehot = jnp.where((k > 0.0) & (pos == jout), 1.0, 0.0)
    out_ref[:] = jnp.dot(onehot, v8_ref[:], preferred_element_type=jnp.float32)


def kernel(boxes, scores):
    probs = jax.nn.sigmoid(scores)
    probs = jnp.clip(probs, _CLAMP, 1.0 - _CLAMP)
    masked = jnp.where(probs > _SCORE_THRESH, probs, -1.0)
    top_vals, top_idx = jax.lax.top_k(masked, _PRE)
    top_boxes = boxes[top_idx]
    v8 = jnp.zeros((_N, 8), jnp.float32)
    v8 = v8.at[:_PRE, :4].set(top_boxes)
    v8 = v8.at[:, 4].set(jnp.pad(top_vals, (0, _N - _PRE), constant_values=-1.0))
    vt = v8.T
    out8 = pl.pallas_call(
        _nms_body,
        out_shape=jax.ShapeDtypeStruct((_POST, 8), jnp.float32),
        scratch_shapes=[pltpu.VMEM((_N, _N), jnp.float32)],
    )(vt, v8)
    return out8[:, :5]
